# baseline (speedup 1.0000x reference)
import functools

import jax
import jax.numpy as jnp
from jax.experimental import pallas as pl

D = 128
NUM_GRAPHS = 64
EPS = 1e-5


def _leaky(x, s):
    return jnp.where(x >= 0, x, s * x)


def _batch_norm(x, g, b):
    mu = jnp.mean(x, axis=0)
    var = jnp.var(x, axis=0)
    return g * (x - mu) / jnp.sqrt(var + EPS) + b


def _masked_batch_norm(x, g, b, mask, cnt):
    w = mask[:, None]
    mu = jnp.sum(x * w, axis=0) / cnt
    var = jnp.sum(w * (x - mu) ** 2, axis=0) / cnt
    return g * (x - mu) / jnp.sqrt(var + EPS) + b


def _l2norm(x):
    n = jnp.sqrt(jnp.sum(x * x, axis=-1, keepdims=True))
    return x / jnp.maximum(n, 1e-12)


def _sage_conv(x, ei, Wl, b, Wr, edge_valid, n):
    src, dst = ei[0], ei[1]
    seg = jnp.where(edge_valid, dst, n)
    s = jax.ops.segment_sum(x[src], seg, num_segments=n)
    cnt = jax.ops.segment_sum(jnp.ones((ei.shape[1],), x.dtype), seg, num_segments=n)
    mean = s / jnp.maximum(cnt, 1.0)[:, None]
    out = mean @ Wl.T + b + x @ Wr.T
    return _l2norm(out)


def _segment_softmax(scores, index, num_segments):
    m = jax.ops.segment_max(scores, index, num_segments=num_segments)
    m = jnp.where(jnp.isfinite(m), m, 0.0)
    e = jnp.exp(scores - m[index])
    den = jax.ops.segment_sum(e, index, num_segments=num_segments)
    return e / (den[index] + 1e-16)


def _match(sc, src, dst, ev, node_valid, n):
    E = src.shape[0]
    BIG = jnp.int32(2**30)
    order = jnp.argsort(jnp.where(ev, -sc, jnp.inf), stable=True)
    rank = jnp.zeros(E, jnp.int32).at[order].set(jnp.arange(E, dtype=jnp.int32))

    def cond(st):
        return st[2]

    def body(st):
        selected, remaining, _ = st
        alive = ev & remaining[src] & remaining[dst]
        key = jnp.where(alive, rank, BIG)
        ms = jax.ops.segment_min(key, src, num_segments=n)
        md = jax.ops.segment_min(key, dst, num_segments=n)
        mnode = jnp.minimum(ms, md)
        win = alive & (key == mnode[src]) & (key == mnode[dst])
        w32 = win.astype(jnp.int32)
        matched = jnp.zeros(n, jnp.int32).at[src].max(w32).at[dst].max(w32)
        remaining = remaining & (matched == 0)
        selected = selected | win
        alive2 = ev & remaining[src] & remaining[dst]
        return selected, remaining, jnp.any(alive2)

    init = (jnp.zeros(E, bool), node_valid, jnp.bool_(True))
    selected, remaining, _ = jax.lax.while_loop(cond, body, init)
    return selected, remaining, order


def _greedy_merge(sc, ei, bat, edge_valid, node_valid, n):
    src, dst = ei[0], ei[1]
    selected, remaining, order = _match(sc, src, dst, edge_valid, node_valid, n)

    sel_ord = selected[order].astype(jnp.int32)
    csum = jnp.cumsum(sel_ord)
    i_total = csum[-1]
    cid_ord = csum - 1
    E = src.shape[0]
    cid = jnp.zeros(E, jnp.int32).at[order].set(cid_ord)
    cluster = jnp.full(n, n, jnp.int32)
    cluster = cluster.at[jnp.where(selected, src, n)].set(cid, mode='drop')
    cluster = cluster.at[jnp.where(selected, dst, n)].set(cid, mode='drop')
    score_arr = jnp.zeros(n, sc.dtype).at[jnp.where(selected, cid, n)].set(sc, mode='drop')

    ridx = remaining.astype(jnp.int32)
    cluster = jnp.where(remaining, i_total + jnp.cumsum(ridx) - ridx, cluster)
    c_new = i_total + jnp.sum(ridx)
    idxs = jnp.arange(n, dtype=jnp.int32)
    new_score = jnp.where(idxs < i_total, score_arr, jnp.ones((), sc.dtype))

    big = jnp.iinfo(jnp.int32).max
    safe_c = jnp.maximum(c_new, 1)
    keyv = jnp.where(edge_valid, cluster[src] * safe_c + cluster[dst], big)
    sk = jnp.sort(keyv)
    prev = jnp.concatenate([jnp.full((1,), -1, sk.dtype), sk[:-1]])
    new_ev = (sk != big) & (sk != prev)
    e0 = jnp.where(new_ev, sk // safe_c, 0)
    e1 = jnp.where(new_ev, sk % safe_c, 0)
    new_ei = jnp.stack([e0, e1]).astype(jnp.int32)

    last = jax.ops.segment_max(idxs, cluster, num_segments=n)
    new_bat = jnp.where(idxs < c_new, bat[jnp.clip(last, 0, n - 1)], jnp.int32(NUM_GRAPHS))
    return cluster, new_score, new_ei, new_ev, new_bat, c_new


def _final_head_kernel(g_ref, w_ref, b_ref, bg_ref, bb_ref, o_ref):
    g = g_ref[...]
    out = jax.lax.dot_general(g, w_ref[...], (((1,), (1,)), ((), ())),
                              preferred_element_type=jnp.float32) + b_ref[...]
    mu = jnp.mean(out, axis=0, keepdims=True)
    var = jnp.mean((out - mu) ** 2, axis=0, keepdims=True)
    out = bg_ref[...] * (out - mu) / jnp.sqrt(var + EPS) + bb_ref[...]
    o_ref[...] = jnp.where(out >= 0, out, 0.01 * out)


def _final_head(g, W, b, bg, bb):
    return pl.pallas_call(
        _final_head_kernel,
        out_shape=jax.ShapeDtypeStruct((NUM_GRAPHS, D), jnp.float32),
    )(g, W, b[None, :], bg[None, :], bb[None, :])


def kernel(x, edge_index, batch, params):
    n = x.shape[0]
    idx_n = jnp.arange(n, dtype=jnp.int32)

    def edge_pool(h, ei, bat, ev, c, wname):
        src, dst = ei[0], ei[1]
        e = jnp.concatenate([h[src], h[dst]], axis=1) @ params[wname + '_W'].T + params[wname + '_b']
        e = e[:, 0]
        seg = jnp.where(ev, dst, n)
        sc = _segment_softmax(e, seg, n) + 0.5
        cluster, new_score, new_ei, new_ev, new_bat, c_new = _greedy_merge(
            sc, ei, bat, ev, idx_n < c, n)
        new_x = jax.ops.segment_sum(h, cluster, num_segments=n)
        new_x = new_x * new_score[:, None]
        return new_x, new_ei, new_bat, new_ev, c_new

    def stage(h, ei, ev, c, conv, lname, bnname, t):
        mask = (idx_n < c).astype(h.dtype)
        cnt = c.astype(h.dtype)
        h = _sage_conv(h, ei, params[conv + '_Wl'], params[conv + '_b'], params[conv + '_Wr'], ev, n)
        h = _masked_batch_norm(h, params[bnname + '_g'], params[bnname + '_b'], mask, cnt)
        h = _leaky(h, 0.01)
        for _ in range(t):
            r = _masked_batch_norm(h, params[lname + 'bn_g'], params[lname + 'bn_b'], mask, cnt)
            r = _sage_conv(r, ei, params[lname + '_Wl'], params[lname + '_b'], params[lname + '_Wr'], ev, n)
            h = h + r
        return h

    c0 = jnp.asarray(n, jnp.int32)
    ev0 = jnp.ones((edge_index.shape[1],), bool)

    h = stage(x, edge_index, ev0, c0, 'sage1', 'l1', 'bn1', 1)
    h, ei, bat, ev, c = edge_pool(h, edge_index, batch, ev0, c0, 'ep1')
    h = stage(h, ei, ev, c, 'sage2', 'l2', 'bn2', 1)
    h, ei, bat, ev, c = edge_pool(h, ei, bat, ev, c, 'ep2')
    h = stage(h, ei, ev, c, 'sage3', 'l3', 'bn3', 1)
    h, ei, bat, ev, c = edge_pool(h, ei, bat, ev, c, 'ep3')
    h = stage(h, ei, ev, c, 'sage4', 'l4', 'bn4', 1)
    g = jax.ops.segment_sum(h, bat, num_segments=NUM_GRAPHS)
    return _final_head(g, params['tr1_W'], params['tr1_b'],
                       params['bn5_g'], params['bn5_b'])


# speedup vs baseline: 29.2027x; 29.2027x over previous
import functools

import jax
import jax.numpy as jnp
from jax import lax
from jax.experimental import pallas as pl
from jax.experimental.pallas import tpu as pltpu
from jax.experimental.pallas import tpu_sc as plsc

D = 128
NUM_GRAPHS = 64
EPS = 1e-5


def _make_match_kernel(E, npad, chunk):
    nvreg = chunk // 16
    nchunk = E // chunk

    def body(s_hbm, t_hbm, nv_hbm, sel_hbm, rem_ref, w_ref, s_buf, t_buf, sel_buf):
        wid = lax.axis_index("s") + lax.axis_index("c") * 16
        lanes = lax.iota(jnp.int32, 16)
        z16 = jnp.zeros((16,), jnp.int32)
        o16 = jnp.ones((16,), jnp.int32)

        @pl.when(wid == 0)
        def _():
            pltpu.sync_copy(nv_hbm, rem_ref)

            def chunk_body(ci, carry):
                base_e = ci * chunk
                pltpu.sync_copy(s_hbm.at[pl.ds(base_e, chunk)], s_buf)
                pltpu.sync_copy(t_hbm.at[pl.ds(base_e, chunk)], t_buf)

                def vreg_body(v, carry2):
                    b = v * 16
                    s = s_buf[pl.ds(b, 16)]
                    t = t_buf[pl.ds(b, 16)]
                    rs = plsc.load_gather(rem_ref, [s])
                    rt = plsc.load_gather(rem_ref, [t])
                    cand = (rs > 0) & (rt > 0)
                    sel_buf[pl.ds(b, 16)] = z16

                    @pl.when(jnp.any(cand))
                    def _():
                        plsc.store_scatter(w_ref, [s], z16, mask=cand)
                        plsc.store_scatter(w_ref, [t], z16, mask=cand)
                        plsc.addupdate_scatter(w_ref, [s], o16, mask=cand)
                        plsc.addupdate_scatter(w_ref, [t], o16, mask=cand)
                        cs = plsc.load_gather(w_ref, [s])
                        ct = plsc.load_gather(w_ref, [t])
                        ok = jnp.where(s == t, cs == 2, (cs == 1) & (ct == 1))
                        conflict = jnp.any(cand & ~ok)

                        @pl.when(~conflict)
                        def _():
                            plsc.store_scatter(rem_ref, [s], z16, mask=cand)
                            plsc.store_scatter(rem_ref, [t], z16, mask=cand)
                            sel_buf[pl.ds(b, 16)] = cand.astype(jnp.int32)

                        @pl.when(conflict)
                        def _():
                            for j in range(16):
                                rs_j = plsc.load_gather(rem_ref, [s])
                                rt_j = plsc.load_gather(rem_ref, [t])
                                win_j = (lanes == j) & (rs_j > 0) & (rt_j > 0)
                                plsc.store_scatter(rem_ref, [s], z16, mask=win_j)
                                plsc.store_scatter(rem_ref, [t], z16, mask=win_j)
                                cur = sel_buf[pl.ds(b, 16)]
                                sel_buf[pl.ds(b, 16)] = cur | win_j.astype(jnp.int32)

                    return carry2

                lax.fori_loop(0, nvreg, vreg_body, 0)
                pltpu.sync_copy(sel_buf, sel_hbm.at[pl.ds(base_e, chunk)])
                return carry

            lax.fori_loop(0, nchunk, chunk_body, 0)

    mesh = plsc.VectorSubcoreMesh(core_axis_name="c", subcore_axis_name="s")
    return pl.kernel(
        body,
        mesh=mesh,
        compiler_params=pltpu.CompilerParams(needs_layout_passes=False),
        out_type=jax.ShapeDtypeStruct((E,), jnp.int32),
        scratch_types=[
            pltpu.VMEM((npad,), jnp.int32),
            pltpu.VMEM((npad,), jnp.int32),
            pltpu.VMEM((chunk,), jnp.int32),
            pltpu.VMEM((chunk,), jnp.int32),
            pltpu.VMEM((chunk,), jnp.int32),
        ],
    )


def _match_sc(sc, src, dst, ev, node_valid, n):
    E = src.shape[0]
    order = jnp.argsort(jnp.where(ev, -sc, jnp.inf), stable=True)
    sent = jnp.int32(n)
    s_ord = jnp.where(ev[order], src[order], sent)
    t_ord = jnp.where(ev[order], dst[order], sent)
    npad = ((n + 16) + 15) // 16 * 16
    nv = jnp.concatenate([node_valid.astype(jnp.int32),
                          jnp.zeros((npad - n,), jnp.int32)])
    chunk = 16000
    sel_ord = _make_match_kernel(E, npad, chunk)(s_ord, t_ord, nv)
    selected = jnp.zeros(E, bool).at[order].set(sel_ord > 0)
    m = jnp.zeros(n, jnp.int32)
    m = m.at[jnp.where(selected, src, n)].set(1, mode='drop')
    m = m.at[jnp.where(selected, dst, n)].set(1, mode='drop')
    remaining = node_valid & (m == 0)
    return selected, remaining, order


def _leaky(x, s):
    return jnp.where(x >= 0, x, s * x)


def _batch_norm(x, g, b):
    mu = jnp.mean(x, axis=0)
    var = jnp.var(x, axis=0)
    return g * (x - mu) / jnp.sqrt(var + EPS) + b


def _masked_batch_norm(x, g, b, mask, cnt):
    w = mask[:, None]
    mu = jnp.sum(x * w, axis=0) / cnt
    var = jnp.sum(w * (x - mu) ** 2, axis=0) / cnt
    return g * (x - mu) / jnp.sqrt(var + EPS) + b


def _l2norm(x):
    n = jnp.sqrt(jnp.sum(x * x, axis=-1, keepdims=True))
    return x / jnp.maximum(n, 1e-12)


def _sage_conv(x, ei, Wl, b, Wr, edge_valid, n):
    src, dst = ei[0], ei[1]
    seg = jnp.where(edge_valid, dst, n)
    s = jax.ops.segment_sum(x[src], seg, num_segments=n)
    cnt = jax.ops.segment_sum(jnp.ones((ei.shape[1],), x.dtype), seg, num_segments=n)
    mean = s / jnp.maximum(cnt, 1.0)[:, None]
    out = mean @ Wl.T + b + x @ Wr.T
    return _l2norm(out)


def _segment_softmax(scores, index, num_segments):
    m = jax.ops.segment_max(scores, index, num_segments=num_segments)
    m = jnp.where(jnp.isfinite(m), m, 0.0)
    e = jnp.exp(scores - m[index])
    den = jax.ops.segment_sum(e, index, num_segments=num_segments)
    return e / (den[index] + 1e-16)


def _match(sc, src, dst, ev, node_valid, n):
    E = src.shape[0]
    BIG = jnp.int32(2**30)
    order = jnp.argsort(jnp.where(ev, -sc, jnp.inf), stable=True)
    rank = jnp.zeros(E, jnp.int32).at[order].set(jnp.arange(E, dtype=jnp.int32))

    def cond(st):
        return st[2]

    def body(st):
        selected, remaining, _ = st
        alive = ev & remaining[src] & remaining[dst]
        key = jnp.where(alive, rank, BIG)
        ms = jax.ops.segment_min(key, src, num_segments=n)
        md = jax.ops.segment_min(key, dst, num_segments=n)
        mnode = jnp.minimum(ms, md)
        win = alive & (key == mnode[src]) & (key == mnode[dst])
        w32 = win.astype(jnp.int32)
        matched = jnp.zeros(n, jnp.int32).at[src].max(w32).at[dst].max(w32)
        remaining = remaining & (matched == 0)
        selected = selected | win
        alive2 = ev & remaining[src] & remaining[dst]
        return selected, remaining, jnp.any(alive2)

    init = (jnp.zeros(E, bool), node_valid, jnp.bool_(True))
    selected, remaining, _ = jax.lax.while_loop(cond, body, init)
    return selected, remaining, order


def _greedy_merge(sc, ei, bat, edge_valid, node_valid, n):
    src, dst = ei[0], ei[1]
    selected, remaining, order = _match_sc(sc, src, dst, edge_valid, node_valid, n)

    sel_ord = selected[order].astype(jnp.int32)
    csum = jnp.cumsum(sel_ord)
    i_total = csum[-1]
    cid_ord = csum - 1
    E = src.shape[0]
    cid = jnp.zeros(E, jnp.int32).at[order].set(cid_ord)
    cluster = jnp.full(n, n, jnp.int32)
    cluster = cluster.at[jnp.where(selected, src, n)].set(cid, mode='drop')
    cluster = cluster.at[jnp.where(selected, dst, n)].set(cid, mode='drop')
    score_arr = jnp.zeros(n, sc.dtype).at[jnp.where(selected, cid, n)].set(sc, mode='drop')

    ridx = remaining.astype(jnp.int32)
    cluster = jnp.where(remaining, i_total + jnp.cumsum(ridx) - ridx, cluster)
    c_new = i_total + jnp.sum(ridx)
    idxs = jnp.arange(n, dtype=jnp.int32)
    new_score = jnp.where(idxs < i_total, score_arr, jnp.ones((), sc.dtype))

    big = jnp.iinfo(jnp.int32).max
    safe_c = jnp.maximum(c_new, 1)
    keyv = jnp.where(edge_valid, cluster[src] * safe_c + cluster[dst], big)
    sk = jnp.sort(keyv)
    prev = jnp.concatenate([jnp.full((1,), -1, sk.dtype), sk[:-1]])
    new_ev = (sk != big) & (sk != prev)
    e0 = jnp.where(new_ev, sk // safe_c, 0)
    e1 = jnp.where(new_ev, sk % safe_c, 0)
    new_ei = jnp.stack([e0, e1]).astype(jnp.int32)

    last = jax.ops.segment_max(idxs, cluster, num_segments=n)
    new_bat = jnp.where(idxs < c_new, bat[jnp.clip(last, 0, n - 1)], jnp.int32(NUM_GRAPHS))
    return cluster, new_score, new_ei, new_ev, new_bat, c_new


def _final_head_kernel(g_ref, w_ref, b_ref, bg_ref, bb_ref, o_ref):
    g = g_ref[...]
    out = jax.lax.dot_general(g, w_ref[...], (((1,), (1,)), ((), ())),
                              preferred_element_type=jnp.float32) + b_ref[...]
    mu = jnp.mean(out, axis=0, keepdims=True)
    var = jnp.mean((out - mu) ** 2, axis=0, keepdims=True)
    out = bg_ref[...] * (out - mu) / jnp.sqrt(var + EPS) + bb_ref[...]
    o_ref[...] = jnp.where(out >= 0, out, 0.01 * out)


def _final_head(g, W, b, bg, bb):
    return pl.pallas_call(
        _final_head_kernel,
        out_shape=jax.ShapeDtypeStruct((NUM_GRAPHS, D), jnp.float32),
    )(g, W, b[None, :], bg[None, :], bb[None, :])


def kernel(x, edge_index, batch, params):
    n = x.shape[0]
    idx_n = jnp.arange(n, dtype=jnp.int32)

    def edge_pool(h, ei, bat, ev, c, wname):
        src, dst = ei[0], ei[1]
        e = jnp.concatenate([h[src], h[dst]], axis=1) @ params[wname + '_W'].T + params[wname + '_b']
        e = e[:, 0]
        seg = jnp.where(ev, dst, n)
        sc = _segment_softmax(e, seg, n) + 0.5
        cluster, new_score, new_ei, new_ev, new_bat, c_new = _greedy_merge(
            sc, ei, bat, ev, idx_n < c, n)
        new_x = jax.ops.segment_sum(h, cluster, num_segments=n)
        new_x = new_x * new_score[:, None]
        return new_x, new_ei, new_bat, new_ev, c_new

    def stage(h, ei, ev, c, conv, lname, bnname, t):
        mask = (idx_n < c).astype(h.dtype)
        cnt = c.astype(h.dtype)
        h = _sage_conv(h, ei, params[conv + '_Wl'], params[conv + '_b'], params[conv + '_Wr'], ev, n)
        h = _masked_batch_norm(h, params[bnname + '_g'], params[bnname + '_b'], mask, cnt)
        h = _leaky(h, 0.01)
        for _ in range(t):
            r = _masked_batch_norm(h, params[lname + 'bn_g'], params[lname + 'bn_b'], mask, cnt)
            r = _sage_conv(r, ei, params[lname + '_Wl'], params[lname + '_b'], params[lname + '_Wr'], ev, n)
            h = h + r
        return h

    c0 = jnp.asarray(n, jnp.int32)
    ev0 = jnp.ones((edge_index.shape[1],), bool)

    h = stage(x, edge_index, ev0, c0, 'sage1', 'l1', 'bn1', 1)
    h, ei, bat, ev, c = edge_pool(h, edge_index, batch, ev0, c0, 'ep1')
    h = stage(h, ei, ev, c, 'sage2', 'l2', 'bn2', 1)
    h, ei, bat, ev, c = edge_pool(h, ei, bat, ev, c, 'ep2')
    h = stage(h, ei, ev, c, 'sage3', 'l3', 'bn3', 1)
    h, ei, bat, ev, c = edge_pool(h, ei, bat, ev, c, 'ep3')
    h = stage(h, ei, ev, c, 'sage4', 'l4', 'bn4', 1)
    g = jax.ops.segment_sum(h, bat, num_segments=NUM_GRAPHS)
    return _final_head(g, params['tr1_W'], params['tr1_b'],
                       params['bn5_g'], params['bn5_b'])
